# Initial kernel scaffold; baseline (speedup 1.0000x reference)
import jax
import jax.numpy as jnp
from jax.experimental import pallas as pl


def kernel(inputs):
    raise NotImplementedError("write your pallas kernel here")


# speedup vs baseline: 36.5263x; 36.5263x over previous
import functools

import jax
import jax.numpy as jnp
from jax import lax
from jax.experimental import pallas as pl
from jax.experimental.pallas import tpu as pltpu
from jax.experimental.pallas import tpu_sc as plsc

B, S, D, C = 4, 8192, 128, 8
NCOL = D * C
LANES = 16
GROUPS = NCOL // LANES
KTOP = 8
CHUNK = 2048
NCHUNK = S // CHUNK

_SORT8 = [(0, 1), (2, 3), (4, 5), (6, 7), (0, 2), (1, 3), (4, 6), (5, 7),
          (1, 2), (5, 6), (0, 4), (1, 5), (2, 6), (3, 7), (2, 4), (3, 5),
          (1, 2), (3, 4), (5, 6)]
_BITONIC8 = [(0, 4), (1, 5), (2, 6), (3, 7), (0, 2), (1, 3), (4, 6), (5, 7),
             (0, 1), (2, 3), (4, 5), (6, 7)]


def _merge_batch(R, N):
    N = list(N)
    for i, j in _SORT8:
        hi = jnp.maximum(N[i], N[j])
        lo = jnp.minimum(N[i], N[j])
        N[i], N[j] = hi, lo
    M = [jnp.maximum(R[i], N[KTOP - 1 - i]) for i in range(KTOP)]
    for i, j in _BITONIC8:
        hi = jnp.maximum(M[i], M[j])
        lo = jnp.minimum(M[i], M[j])
        M[i], M[j] = hi, lo
    return tuple(M)


def _sc_body(x_hbm, out_hbm, buf0, buf1, obuf, sem0, sem1):
    info = plsc.get_sparse_core_info()
    nc = info.num_cores
    wid = lax.axis_index("s") * nc + lax.axis_index("c")
    bufs = (buf0, buf1)
    sems = (sem0, sem1)
    ntasks = B * GROUPS // (nc * info.num_subcores)

    def chunk_copy(task, c, slot):
        t = wid * ntasks + task
        b = t // GROUPS
        col0 = (t % GROUPS) * LANES
        return pltpu.async_copy(
            x_hbm.at[b, pl.ds(c * CHUNK, CHUNK), pl.ds(col0, LANES)],
            bufs[slot], sems[slot])

    def compute_chunk(slot, R):
        buf = bufs[slot]

        def body(k, R):
            rows = tuple(buf[k * KTOP + i] for i in range(KTOP))
            return _merge_batch(R, rows)

        return lax.fori_loop(0, CHUNK // KTOP, body, R)

    cp = chunk_copy(0, 0, 0)
    for task in range(ntasks):
        R = tuple(jnp.full((LANES,), -jnp.inf, jnp.float32)
                  for _ in range(KTOP))
        for c in range(NCHUNK):
            slot = c % 2
            cp.wait()
            nc_, nt = (c + 1, task) if c + 1 < NCHUNK else (0, task + 1)
            if nt < ntasks:
                cp = chunk_copy(nt, nc_, 1 - slot)
            R = compute_chunk(slot, R)
        for j in range(KTOP):
            obuf[j] = R[j]
        t = wid * ntasks + task
        b = t // GROUPS
        col0 = (t % GROUPS) * LANES
        pltpu.sync_copy(obuf, out_hbm.at[b, :, pl.ds(col0, LANES)])


def kernel(inputs):
    x3 = inputs.reshape(B, S, NCOL)
    mesh = plsc.VectorSubcoreMesh(core_axis_name="c", subcore_axis_name="s")
    run = functools.partial(
        pl.kernel, mesh=mesh,
        compiler_params=pltpu.CompilerParams(use_tc_tiling_on_sc=False),
        out_type=jax.ShapeDtypeStruct((B, KTOP, NCOL), jnp.float32),
        scratch_types=[
            pltpu.VMEM((CHUNK, LANES), jnp.float32),
            pltpu.VMEM((CHUNK, LANES), jnp.float32),
            pltpu.VMEM((KTOP, LANES), jnp.float32),
            pltpu.SemaphoreType.DMA,
            pltpu.SemaphoreType.DMA,
        ],
    )(_sc_body)
    return run(x3).reshape(B, KTOP, D, C)
